# baseline (speedup 1.0000x reference)
import jax
import jax.numpy as jnp
from jax import lax
from jax.experimental import pallas as pl
from jax.experimental.pallas import tpu as pltpu
from jax.experimental.pallas import tpu_sc as plsc

K = 512
ROWS = 64
COLS = 8192
LANES = 16
NUM_CORES = 2
NUM_SUBCORES = 16
NUM_WORKERS = NUM_CORES * NUM_SUBCORES
ROWS_PER_WORKER = ROWS // NUM_WORKERS
VREGS_PER_ROW = COLS // LANES
UNROLL = 32


def _sortable_key(xv):
    b = lax.bitcast_convert_type(xv, jnp.int32)
    return b ^ (jnp.right_shift(b, 31) & jnp.int32(0x7FFFFFFF))


def _hist_zero(hist_ref):
    zeros = jnp.zeros((LANES,), jnp.int32)
    for j in range(256 // LANES):
        hist_ref[pl.ds(j * LANES, LANES)] = zeros


def _hist_scan(hist_ref, k_rem):
    iota = lax.iota(jnp.int32, LANES)

    def cond(carry):
        found, j, rb_star, above, e, cum = carry
        return jnp.logical_and(found == 0, j < 256 // LANES)

    def body(carry):
        found, j, rb_star, above, e, cum = carry
        h = hist_ref[pl.ds(j * LANES, LANES)]
        c = plsc.cumsum(h)
        test = (cum + c) >= k_rem
        ffs = jnp.max(plsc.all_reduce_ffs(test))
        hit = ffs < LANES
        above_in = jnp.sum(jnp.where(iota < ffs, h, 0))
        e_in = jnp.sum(jnp.where(iota == ffs, h, 0))
        rb_star = jnp.where(hit, j * LANES + ffs, rb_star)
        above = jnp.where(hit, cum + above_in, above)
        e = jnp.where(hit, e_in, e)
        found = jnp.where(hit, jnp.int32(1), found)
        cum = cum + jnp.max(c)
        return found, j + 1, rb_star, above, e, cum

    z = jnp.int32(0)
    _, _, rb_star, above, e, _ = lax.while_loop(
        cond, body, (z, z, z, z, z, z))
    return rb_star, above, e


def _topk_body(x_hbm, out_hbm, rows_ref, keys_ref, hist_ref,
               sem_in, sem_out):
    wid = lax.axis_index("s") * NUM_CORES + lax.axis_index("c")
    base_row = wid * ROWS_PER_WORKER
    for r in range(ROWS_PER_WORKER):
        pltpu.async_copy(x_hbm.at[base_row + r],
                         rows_ref.at[pl.ds(r * COLS, COLS)], sem_in)

    ones = jnp.ones((LANES,), jnp.int32)
    iota = lax.iota(jnp.int32, LANES)
    zero_f = jnp.float32(0.0)

    def per_row(r, _):
        roff = r * COLS
        pltpu.make_async_copy(
            x_hbm.at[base_row],
            rows_ref.at[pl.ds(roff, COLS)], sem_in).wait()

        _hist_zero(hist_ref)

        @plsc.parallel_loop(0, VREGS_PER_ROW, unroll=UNROLL)
        def l0_body(i):
            sl = pl.ds(roff + i * LANES, LANES)
            skey = _sortable_key(rows_ref[sl])
            keys_ref[sl] = skey
            rb = jnp.int32(127) - jnp.right_shift(skey, 24)
            plsc.addupdate_scatter(hist_ref, [rb], ones)

        rb_star, above, e0 = _hist_scan(hist_ref, jnp.int32(K))

        def per_level(carry):
            lvl, thr, k_rem, _ = carry
            shift = 24 - 8 * lvl
            pmask = lax.shift_left(jnp.int32(-1), shift + 8)

            _hist_zero(hist_ref)

            @plsc.parallel_loop(0, VREGS_PER_ROW, unroll=UNROLL)
            def ln_body(i):
                sl = pl.ds(roff + i * LANES, LANES)
                k = keys_ref[sl]
                pm = (k & pmask) == thr
                rb = jnp.int32(255) - (jnp.right_shift(k, shift) & 255)
                plsc.addupdate_scatter(hist_ref, [rb], ones, mask=pm)

            rb_star, above, e = _hist_scan(hist_ref, k_rem)
            k_rem = k_rem - above
            thr = thr | lax.shift_left(jnp.int32(255) - rb_star, shift)
            return lvl + 1, thr, k_rem, e

        def level_cond(carry):
            lvl, thr, k_rem, e = carry
            return jnp.logical_and(lvl < 4, e != k_rem)

        thr = lax.shift_left(jnp.int32(127) - rb_star, 24)
        k_rem = jnp.int32(K) - above
        _, thr, k_rem, e = lax.while_loop(
            level_cond, per_level, (jnp.int32(1), thr, k_rem, e0))

        def fast_sweep():
            thr_b = thr ^ (jnp.right_shift(thr, 31) & jnp.int32(0x7FFFFFFF))
            thr_f = lax.bitcast_convert_type(
                jnp.broadcast_to(thr_b, (LANES,)), jnp.float32)

            @plsc.parallel_loop(0, VREGS_PER_ROW, unroll=UNROLL)
            def body(i):
                sl = pl.ds(roff + i * LANES, LANES)
                xv = rows_ref[sl]
                rows_ref[sl] = jnp.where(xv >= thr_f, xv, zero_f)

        def tie_sweep():
            def body(i, run):
                for u in range(UNROLL):
                    sl = pl.ds(roff + (i * UNROLL + u) * LANES, LANES)
                    k = keys_ref[sl]
                    xv = rows_ref[sl]
                    eq = k == thr
                    m = eq.astype(jnp.int32)
                    pc = plsc.cumsum(m)
                    keep = (k > thr) | (eq & ((run + pc) <= k_rem))
                    rows_ref[sl] = jnp.where(keep, xv, zero_f)
                    run = run + jnp.max(pc)
                return run

            lax.fori_loop(0, VREGS_PER_ROW // UNROLL, body, jnp.int32(0))

        lax.cond(e == k_rem, fast_sweep, tie_sweep)

        pltpu.async_copy(rows_ref.at[pl.ds(roff, COLS)],
                         out_hbm.at[base_row + r], sem_out)
        return 0

    lax.fori_loop(0, ROWS_PER_WORKER, per_row, 0)

    for r in range(ROWS_PER_WORKER):
        pltpu.make_async_copy(
            rows_ref.at[pl.ds(r * COLS, COLS)],
            out_hbm.at[base_row + r], sem_out).wait()


@jax.jit
def kernel(x):
    mesh = plsc.VectorSubcoreMesh(
        core_axis_name="c", subcore_axis_name="s",
        num_cores=NUM_CORES, num_subcores=NUM_SUBCORES)
    return pl.kernel(
        _topk_body,
        out_type=jax.ShapeDtypeStruct((ROWS, COLS), jnp.float32),
        mesh=mesh,
        scratch_types=[
            pltpu.VMEM((ROWS_PER_WORKER * COLS,), jnp.float32),
            pltpu.VMEM((ROWS_PER_WORKER * COLS,), jnp.int32),
            pltpu.VMEM((256,), jnp.int32),
            pltpu.SemaphoreType.DMA,
            pltpu.SemaphoreType.DMA,
        ],
        compiler_params=pltpu.CompilerParams(needs_layout_passes=False),
    )(x)


# speedup vs baseline: 1.0353x; 1.0353x over previous
import jax
import jax.numpy as jnp
from jax import lax
from jax.experimental import pallas as pl
from jax.experimental.pallas import tpu as pltpu
from jax.experimental.pallas import tpu_sc as plsc

K = 512
ROWS = 64
COLS = 8192
LANES = 16
NUM_CORES = 2
NUM_SUBCORES = 16
NUM_WORKERS = NUM_CORES * NUM_SUBCORES
ROWS_PER_WORKER = ROWS // NUM_WORKERS
VREGS_PER_ROW = COLS // LANES
UNROLL = 16


def _sortable_key(xv):
    b = lax.bitcast_convert_type(xv, jnp.int32)
    return b ^ (jnp.right_shift(b, 31) & jnp.int32(0x7FFFFFFF))


def _hist_zero(hist_ref):
    zeros = jnp.zeros((LANES,), jnp.int32)
    for j in range(256 // LANES):
        hist_ref[pl.ds(j * LANES, LANES)] = zeros


def _hist_scan(hist_ref, k_rem):
    iota = lax.iota(jnp.int32, LANES)

    def cond(carry):
        found, j, rb_star, above, e, cum = carry
        return jnp.logical_and(found == 0, j < 256 // LANES)

    def body(carry):
        found, j, rb_star, above, e, cum = carry
        h = hist_ref[pl.ds(j * LANES, LANES)]
        c = plsc.cumsum(h)
        test = (cum + c) >= k_rem
        ffs = jnp.max(plsc.all_reduce_ffs(test))
        hit = ffs < LANES
        above_in = jnp.sum(jnp.where(iota < ffs, h, 0))
        e_in = jnp.sum(jnp.where(iota == ffs, h, 0))
        rb_star = jnp.where(hit, j * LANES + ffs, rb_star)
        above = jnp.where(hit, cum + above_in, above)
        e = jnp.where(hit, e_in, e)
        found = jnp.where(hit, jnp.int32(1), found)
        cum = cum + jnp.max(c)
        return found, j + 1, rb_star, above, e, cum

    z = jnp.int32(0)
    _, _, rb_star, above, e, _ = lax.while_loop(
        cond, body, (z, z, z, z, z, z))
    return rb_star, above, e


def _topk_body(x_hbm, out_hbm, rows_ref, keys_ref, hist_ref,
               sem_in, sem_out):
    wid = lax.axis_index("s") * NUM_CORES + lax.axis_index("c")
    base_row = wid * ROWS_PER_WORKER
    for r in range(ROWS_PER_WORKER):
        pltpu.async_copy(x_hbm.at[base_row + r],
                         rows_ref.at[pl.ds(r * COLS, COLS)], sem_in)

    ones = jnp.ones((LANES,), jnp.int32)
    iota = lax.iota(jnp.int32, LANES)
    zero_f = jnp.float32(0.0)

    def per_row(r, _):
        roff = r * COLS
        pltpu.make_async_copy(
            x_hbm.at[base_row],
            rows_ref.at[pl.ds(roff, COLS)], sem_in).wait()

        _hist_zero(hist_ref)

        @plsc.parallel_loop(0, VREGS_PER_ROW, unroll=UNROLL)
        def l0_body(i):
            sl = pl.ds(roff + i * LANES, LANES)
            skey = _sortable_key(rows_ref[sl])
            keys_ref[sl] = skey
            rb = jnp.int32(127) - jnp.right_shift(skey, 24)
            plsc.addupdate_scatter(hist_ref, [rb], ones)

        rb_star, above, e0 = _hist_scan(hist_ref, jnp.int32(K))

        def per_level(carry):
            lvl, thr, k_rem, _ = carry
            shift = 24 - 8 * lvl
            pmask = lax.shift_left(jnp.int32(-1), shift + 8)

            _hist_zero(hist_ref)

            @plsc.parallel_loop(0, VREGS_PER_ROW, unroll=UNROLL)
            def ln_body(i):
                sl = pl.ds(roff + i * LANES, LANES)
                k = keys_ref[sl]
                pm = (k & pmask) == thr
                rb = jnp.int32(255) - (jnp.right_shift(k, shift) & 255)
                plsc.addupdate_scatter(hist_ref, [rb], ones, mask=pm)

            rb_star, above, e = _hist_scan(hist_ref, k_rem)
            k_rem = k_rem - above
            thr = thr | lax.shift_left(jnp.int32(255) - rb_star, shift)
            return lvl + 1, thr, k_rem, e

        def level_cond(carry):
            lvl, thr, k_rem, e = carry
            return jnp.logical_and(lvl < 4, e != k_rem)

        thr = lax.shift_left(jnp.int32(127) - rb_star, 24)
        k_rem = jnp.int32(K) - above
        _, thr, k_rem, e = lax.while_loop(
            level_cond, per_level, (jnp.int32(1), thr, k_rem, e0))

        def fast_sweep():
            thr_b = thr ^ (jnp.right_shift(thr, 31) & jnp.int32(0x7FFFFFFF))
            thr_f = lax.bitcast_convert_type(
                jnp.broadcast_to(thr_b, (LANES,)), jnp.float32)

            @plsc.parallel_loop(0, VREGS_PER_ROW, unroll=UNROLL)
            def body(i):
                sl = pl.ds(roff + i * LANES, LANES)
                xv = rows_ref[sl]
                rows_ref[sl] = jnp.where(xv >= thr_f, xv, zero_f)

        def tie_sweep():
            def body(i, run):
                for u in range(UNROLL):
                    sl = pl.ds(roff + (i * UNROLL + u) * LANES, LANES)
                    k = keys_ref[sl]
                    xv = rows_ref[sl]
                    eq = k == thr
                    m = eq.astype(jnp.int32)
                    pc = plsc.cumsum(m)
                    keep = (k > thr) | (eq & ((run + pc) <= k_rem))
                    rows_ref[sl] = jnp.where(keep, xv, zero_f)
                    run = run + jnp.max(pc)
                return run

            lax.fori_loop(0, VREGS_PER_ROW // UNROLL, body, jnp.int32(0))

        lax.cond(e == k_rem, fast_sweep, tie_sweep)

        pltpu.async_copy(rows_ref.at[pl.ds(roff, COLS)],
                         out_hbm.at[base_row + r], sem_out)
        return 0

    lax.fori_loop(0, ROWS_PER_WORKER, per_row, 0)

    for r in range(ROWS_PER_WORKER):
        pltpu.make_async_copy(
            rows_ref.at[pl.ds(r * COLS, COLS)],
            out_hbm.at[base_row + r], sem_out).wait()


@jax.jit
def kernel(x):
    mesh = plsc.VectorSubcoreMesh(
        core_axis_name="c", subcore_axis_name="s",
        num_cores=NUM_CORES, num_subcores=NUM_SUBCORES)
    return pl.kernel(
        _topk_body,
        out_type=jax.ShapeDtypeStruct((ROWS, COLS), jnp.float32),
        mesh=mesh,
        scratch_types=[
            pltpu.VMEM((ROWS_PER_WORKER * COLS,), jnp.float32),
            pltpu.VMEM((ROWS_PER_WORKER * COLS,), jnp.int32),
            pltpu.VMEM((256,), jnp.int32),
            pltpu.SemaphoreType.DMA,
            pltpu.SemaphoreType.DMA,
        ],
        compiler_params=pltpu.CompilerParams(needs_layout_passes=False),
    )(x)
